# Initial kernel scaffold; baseline (speedup 1.0000x reference)
import jax
import jax.numpy as jnp
from jax.experimental import pallas as pl


def kernel(points, planes):
    raise NotImplementedError("write your pallas kernel here")


# speedup vs baseline: 9.0291x; 9.0291x over previous
import jax
import jax.numpy as jnp
from jax.experimental import pallas as pl

_THR = 0.01
_NPAD = 50048
_NL = _NPAD // 8


def _fit_kernel(pts_ref, pp_ref, mom_ref, bits_ref):
    pts3 = pts_ref[0, 0:3, :]
    lm = pts_ref[0, 3:4, :]
    un = pp_ref[0, :, 0:3]
    dist = pp_ref[0, :, 3:4]
    valid = pp_ref[0, :, 4:5]
    d = jax.lax.dot_general(un, pts3, (((1,), (0,)), ((), ())),
                            preferred_element_type=jnp.float32)
    pd = jnp.abs(d + dist)
    maskf = ((pd < _THR) & (valid > 0.5) & (lm < 0.5)).astype(jnp.float32)

    x = pts3[0:1, :]
    y = pts3[1:2, :]
    z = pts3[2:3, :]
    cnt = jnp.sum(maskf, axis=1, keepdims=True)
    sx = jnp.sum(maskf * x, axis=1, keepdims=True)
    sy = jnp.sum(maskf * y, axis=1, keepdims=True)
    sz = jnp.sum(maskf * z, axis=1, keepdims=True)
    cd = jnp.maximum(cnt, 1.0)
    cx = sx / cd
    cy = sy / cd
    cz = sz / cd
    bx = ((x - cx) * maskf).astype(jnp.bfloat16).astype(jnp.float32)
    by = ((y - cy) * maskf).astype(jnp.bfloat16).astype(jnp.float32)
    bz = ((z - cz) * maskf).astype(jnp.bfloat16).astype(jnp.float32)
    cxx = jnp.sum(bx * bx, axis=1, keepdims=True)
    cyy = jnp.sum(by * by, axis=1, keepdims=True)
    czz = jnp.sum(bz * bz, axis=1, keepdims=True)
    cxy = jnp.sum(bx * by, axis=1, keepdims=True)
    cxz = jnp.sum(bx * bz, axis=1, keepdims=True)
    cyz = jnp.sum(by * bz, axis=1, keepdims=True)
    zpad = jnp.zeros((32, 118), jnp.float32)
    mom_ref[0] = jnp.concatenate(
        [cnt, sx, sy, sz, cxx, cyy, czz, cxy, cxz, cyz, zpad], axis=1)

    i = jax.lax.broadcasted_iota(jnp.int32, (1, 32), 1)
    pw_lo = jnp.where(i < 16, jnp.left_shift(1, jnp.minimum(i, 15)),
                      0).astype(jnp.float32)
    pw_hi = jnp.where(i >= 16, jnp.left_shift(1, jnp.maximum(i - 16, 0)),
                      0).astype(jnp.float32)
    lo = jax.lax.dot_general(pw_lo, maskf, (((1,), (0,)), ((), ())),
                             preferred_element_type=jnp.float32)
    hi = jax.lax.dot_general(pw_hi, maskf, (((1,), (0,)), ((), ())),
                             preferred_element_type=jnp.float32)
    zrows = jnp.zeros((6, lo.shape[1]), jnp.float32)
    bits_ref[0] = jnp.concatenate([lo, hi, zrows], axis=0)


def _sweep_kernel(pts_ref, bits_ref, pp_ref, proj_ref, disp_ref):
    X = pts_ref[0, 0:8, :]
    Y = pts_ref[0, 8:16, :]
    Z = pts_ref[0, 16:24, :]
    bits = bits_ref[0]
    px, py, pz = X, Y, Z
    for m in range(32):
        am = (jax.lax.shift_right_logical(bits, jnp.uint32(m))
              & jnp.uint32(1)).astype(jnp.float32)
        rnx = pp_ref[0, m:m + 1, 0:1]
        rny = pp_ref[0, m:m + 1, 1:2]
        rnz = pp_ref[0, m:m + 1, 2:3]
        rd = pp_ref[0, m:m + 1, 3:4]
        dots = rnx * px + rny * py + rnz * pz + rd
        t = am * dots
        px = px - rnx * t
        py = py - rny * t
        pz = pz - rnz * t
    proj_ref[0, 0:8, :] = px
    proj_ref[0, 8:16, :] = py
    proj_ref[0, 16:24, :] = pz
    disp_ref[0, 0:8, :] = px - X
    disp_ref[0, 8:16, :] = py - Y
    disp_ref[0, 16:24, :] = pz - Z


def kernel(points, planes):
    B, N, _ = points.shape
    M = planes.shape[1]
    pad = _NPAD - N

    pts_t = jnp.transpose(points, (0, 2, 1))
    pts_tp = jnp.pad(pts_t, ((0, 0), (0, 0), (0, pad)))
    lanemask = jnp.concatenate(
        [jnp.zeros((B, 1, N), jnp.float32), jnp.ones((B, 1, pad), jnp.float32)],
        axis=2)
    pts_a = jnp.concatenate(
        [pts_tp, lanemask, jnp.zeros((B, 4, _NPAD), jnp.float32)], axis=1)

    normals = planes[:, :, :3]
    dists = planes[:, :, 3]
    norm_mag = jnp.linalg.norm(normals, axis=2)
    valid = norm_mag > 1e-6
    un = normals / jnp.maximum(norm_mag, 1e-12)[..., None]
    pp_a = jnp.concatenate(
        [un, dists[..., None], valid.astype(jnp.float32)[..., None],
         jnp.zeros((B, M, 123), jnp.float32)], axis=2)

    mom, bits_f = pl.pallas_call(
        _fit_kernel,
        grid=(B,),
        in_specs=[
            pl.BlockSpec((1, 8, _NPAD), lambda b: (b, 0, 0)),
            pl.BlockSpec((1, M, 128), lambda b: (b, 0, 0)),
        ],
        out_specs=[
            pl.BlockSpec((1, M, 128), lambda b: (b, 0, 0)),
            pl.BlockSpec((1, 8, _NPAD), lambda b: (b, 0, 0)),
        ],
        out_shape=[
            jax.ShapeDtypeStruct((B, M, 128), jnp.float32),
            jax.ShapeDtypeStruct((B, 8, _NPAD), jnp.float32),
        ],
    )(pts_a, pp_a)

    cnt = mom[:, :, 0]
    s = mom[:, :, 1:4]
    c = s / jnp.maximum(cnt, 1.0)[..., None]
    denom = jnp.maximum(1.0, cnt - 1.0)
    xx, yy, zz = mom[:, :, 4], mom[:, :, 5], mom[:, :, 6]
    xy, xz, yz = mom[:, :, 7], mom[:, :, 8], mom[:, :, 9]
    s2 = jnp.stack([
        jnp.stack([xx, xy, xz], -1),
        jnp.stack([xy, yy, yz], -1),
        jnp.stack([xz, yz, zz], -1)], -2)
    cov = s2 / denom[..., None, None] + 1e-6 * jnp.eye(3, dtype=jnp.float32)
    do_fit = cnt >= 3.0
    dummy = jnp.diag(jnp.array([3.0, 2.0, 1.0], dtype=jnp.float32))
    cov_safe = jnp.where(do_fit[..., None, None], cov, dummy)
    _, _, vh = jnp.linalg.svd(cov_safe)
    rn = vh[..., :, 2]
    dotn = jnp.sum(rn * un, -1)
    rn = jnp.where((dotn < 0.0)[..., None], -rn, rn)
    rd = -jnp.sum(c * rn, -1)
    fitf = do_fit.astype(jnp.float32)
    rn_eff = rn * fitf[..., None]
    rd_eff = rd * fitf
    pp_b = jnp.concatenate(
        [rn_eff, rd_eff[..., None], jnp.zeros((B, M, 124), jnp.float32)],
        axis=2)

    bits_u = (bits_f[:, 0, :].astype(jnp.uint32)
              | (bits_f[:, 1, :].astype(jnp.uint32) << 16))
    bits_b = bits_u.reshape(B, 8, _NL)
    pts_b = pts_tp.reshape(B, 24, _NL)

    proj_r, disp_r = pl.pallas_call(
        _sweep_kernel,
        grid=(B,),
        in_specs=[
            pl.BlockSpec((1, 24, _NL), lambda b: (b, 0, 0)),
            pl.BlockSpec((1, 8, _NL), lambda b: (b, 0, 0)),
            pl.BlockSpec((1, M, 128), lambda b: (b, 0, 0)),
        ],
        out_specs=[
            pl.BlockSpec((1, 24, _NL), lambda b: (b, 0, 0)),
            pl.BlockSpec((1, 24, _NL), lambda b: (b, 0, 0)),
        ],
        out_shape=[
            jax.ShapeDtypeStruct((B, 24, _NL), jnp.float32),
            jax.ShapeDtypeStruct((B, 24, _NL), jnp.float32),
        ],
    )(pts_b, bits_b, pp_b)

    proj = jnp.transpose(proj_r.reshape(B, 3, _NPAD), (0, 2, 1))[:, :N, :]
    disp = jnp.transpose(disp_r.reshape(B, 3, _NPAD), (0, 2, 1))[:, :N, :]
    return proj, disp


# speedup vs baseline: 14.1281x; 1.5647x over previous
import jax
import jax.numpy as jnp
from jax.experimental import pallas as pl

_THR = 0.01
_NPAD = 50048
_NL = _NPAD // 8


def _fit_kernel(pts_ref, pp_ref, mom_ref, bits_ref):
    pts3 = pts_ref[0, 0:3, :]
    lm = pts_ref[0, 3:4, :]
    un = pp_ref[0, :, 0:3]
    dist = pp_ref[0, :, 3:4]
    valid = pp_ref[0, :, 4:5]
    d = jax.lax.dot_general(un, pts3, (((1,), (0,)), ((), ())),
                            preferred_element_type=jnp.float32)
    pd = jnp.abs(d + dist)
    maskf = ((pd < _THR) & (valid > 0.5) & (lm < 0.5)).astype(jnp.float32)

    x = pts3[0:1, :]
    y = pts3[1:2, :]
    z = pts3[2:3, :]
    cnt = jnp.sum(maskf, axis=1, keepdims=True)
    sx = jnp.sum(maskf * x, axis=1, keepdims=True)
    sy = jnp.sum(maskf * y, axis=1, keepdims=True)
    sz = jnp.sum(maskf * z, axis=1, keepdims=True)
    cd = jnp.maximum(cnt, 1.0)
    cx = sx / cd
    cy = sy / cd
    cz = sz / cd
    bx = ((x - cx) * maskf).astype(jnp.bfloat16).astype(jnp.float32)
    by = ((y - cy) * maskf).astype(jnp.bfloat16).astype(jnp.float32)
    bz = ((z - cz) * maskf).astype(jnp.bfloat16).astype(jnp.float32)
    cxx = jnp.sum(bx * bx, axis=1, keepdims=True)
    cyy = jnp.sum(by * by, axis=1, keepdims=True)
    czz = jnp.sum(bz * bz, axis=1, keepdims=True)
    cxy = jnp.sum(bx * by, axis=1, keepdims=True)
    cxz = jnp.sum(bx * bz, axis=1, keepdims=True)
    cyz = jnp.sum(by * bz, axis=1, keepdims=True)
    zpad = jnp.zeros((32, 118), jnp.float32)
    mom_ref[0] = jnp.concatenate(
        [cnt, sx, sy, sz, cxx, cyy, czz, cxy, cxz, cyz, zpad], axis=1)

    i = jax.lax.broadcasted_iota(jnp.int32, (1, 32), 1)
    pw_lo = jnp.where(i < 16, jnp.left_shift(1, jnp.minimum(i, 15)),
                      0).astype(jnp.float32)
    pw_hi = jnp.where(i >= 16, jnp.left_shift(1, jnp.maximum(i - 16, 0)),
                      0).astype(jnp.float32)
    lo = jax.lax.dot_general(pw_lo, maskf, (((1,), (0,)), ((), ())),
                             preferred_element_type=jnp.float32)
    hi = jax.lax.dot_general(pw_hi, maskf, (((1,), (0,)), ((), ())),
                             preferred_element_type=jnp.float32)
    zrows = jnp.zeros((6, lo.shape[1]), jnp.float32)
    bits_ref[0] = jnp.concatenate([lo, hi, zrows], axis=0)


def _sweep_kernel(pts_ref, bits_ref, pp_ref, proj_ref, disp_ref):
    X = pts_ref[0, 0:8, :]
    Y = pts_ref[0, 8:16, :]
    Z = pts_ref[0, 16:24, :]
    bits = bits_ref[0]
    px, py, pz = X, Y, Z
    for m in range(32):
        am = (jax.lax.shift_right_logical(bits, jnp.uint32(m))
              & jnp.uint32(1)).astype(jnp.float32)
        rnx = pp_ref[0, m:m + 1, 0:1]
        rny = pp_ref[0, m:m + 1, 1:2]
        rnz = pp_ref[0, m:m + 1, 2:3]
        rd = pp_ref[0, m:m + 1, 3:4]
        dots = rnx * px + rny * py + rnz * pz + rd
        t = am * dots
        px = px - rnx * t
        py = py - rny * t
        pz = pz - rnz * t
    proj_ref[0, 0:8, :] = px
    proj_ref[0, 8:16, :] = py
    proj_ref[0, 16:24, :] = pz
    disp_ref[0, 0:8, :] = px - X
    disp_ref[0, 8:16, :] = py - Y
    disp_ref[0, 16:24, :] = pz - Z


def kernel(points, planes):
    B, N, _ = points.shape
    M = planes.shape[1]
    pad = _NPAD - N

    pts_t = jnp.transpose(points, (0, 2, 1))
    pts_tp = jnp.pad(pts_t, ((0, 0), (0, 0), (0, pad)))
    lanemask = jnp.concatenate(
        [jnp.zeros((B, 1, N), jnp.float32), jnp.ones((B, 1, pad), jnp.float32)],
        axis=2)
    pts_a = jnp.concatenate(
        [pts_tp, lanemask, jnp.zeros((B, 4, _NPAD), jnp.float32)], axis=1)

    normals = planes[:, :, :3]
    dists = planes[:, :, 3]
    norm_mag = jnp.linalg.norm(normals, axis=2)
    valid = norm_mag > 1e-6
    un = normals / jnp.maximum(norm_mag, 1e-12)[..., None]
    pp_a = jnp.concatenate(
        [un, dists[..., None], valid.astype(jnp.float32)[..., None],
         jnp.zeros((B, M, 123), jnp.float32)], axis=2)

    mom, bits_f = pl.pallas_call(
        _fit_kernel,
        grid=(B,),
        in_specs=[
            pl.BlockSpec((1, 8, _NPAD), lambda b: (b, 0, 0)),
            pl.BlockSpec((1, M, 128), lambda b: (b, 0, 0)),
        ],
        out_specs=[
            pl.BlockSpec((1, M, 128), lambda b: (b, 0, 0)),
            pl.BlockSpec((1, 8, _NPAD), lambda b: (b, 0, 0)),
        ],
        out_shape=[
            jax.ShapeDtypeStruct((B, M, 128), jnp.float32),
            jax.ShapeDtypeStruct((B, 8, _NPAD), jnp.float32),
        ],
    )(pts_a, pp_a)

    cnt = mom[:, :, 0]
    s = mom[:, :, 1:4]
    c = s / jnp.maximum(cnt, 1.0)[..., None]
    denom = jnp.maximum(1.0, cnt - 1.0)
    xx, yy, zz = mom[:, :, 4], mom[:, :, 5], mom[:, :, 6]
    xy, xz, yz = mom[:, :, 7], mom[:, :, 8], mom[:, :, 9]
    s2 = jnp.stack([
        jnp.stack([xx, xy, xz], -1),
        jnp.stack([xy, yy, yz], -1),
        jnp.stack([xz, yz, zz], -1)], -2)
    cov = s2 / denom[..., None, None] + 1e-6 * jnp.eye(3, dtype=jnp.float32)
    do_fit = cnt >= 3.0
    dummy = jnp.diag(jnp.array([3.0, 2.0, 1.0], dtype=jnp.float32))
    cov_safe = jnp.where(do_fit[..., None, None], cov, dummy)
    _, vee = jnp.linalg.eigh(cov_safe)
    rn = vee[..., 2, ::-1]
    dotn = jnp.sum(rn * un, -1)
    rn = jnp.where((dotn < 0.0)[..., None], -rn, rn)
    rd = -jnp.sum(c * rn, -1)
    fitf = do_fit.astype(jnp.float32)
    rn_eff = rn * fitf[..., None]
    rd_eff = rd * fitf
    pp_b = jnp.concatenate(
        [rn_eff, rd_eff[..., None], jnp.zeros((B, M, 124), jnp.float32)],
        axis=2)

    bits_u = (bits_f[:, 0, :].astype(jnp.uint32)
              | (bits_f[:, 1, :].astype(jnp.uint32) << 16))
    bits_b = bits_u.reshape(B, 8, _NL)
    pts_b = pts_tp.reshape(B, 24, _NL)

    proj_r, disp_r = pl.pallas_call(
        _sweep_kernel,
        grid=(B,),
        in_specs=[
            pl.BlockSpec((1, 24, _NL), lambda b: (b, 0, 0)),
            pl.BlockSpec((1, 8, _NL), lambda b: (b, 0, 0)),
            pl.BlockSpec((1, M, 128), lambda b: (b, 0, 0)),
        ],
        out_specs=[
            pl.BlockSpec((1, 24, _NL), lambda b: (b, 0, 0)),
            pl.BlockSpec((1, 24, _NL), lambda b: (b, 0, 0)),
        ],
        out_shape=[
            jax.ShapeDtypeStruct((B, 24, _NL), jnp.float32),
            jax.ShapeDtypeStruct((B, 24, _NL), jnp.float32),
        ],
    )(pts_b, bits_b, pp_b)

    proj = jnp.transpose(proj_r.reshape(B, 3, _NPAD), (0, 2, 1))[:, :N, :]
    disp = jnp.transpose(disp_r.reshape(B, 3, _NPAD), (0, 2, 1))[:, :N, :]
    return proj, disp


# speedup vs baseline: 14.5708x; 1.0313x over previous
import functools

import jax
import jax.numpy as jnp
from jax.experimental import pallas as pl

_THR = 0.01
_NPAD = 50048
_NL = _NPAD // 8


def _fit_kernel(pts_ref, pp_ref, mom_ref, bits_ref, *, n_valid):
    pts3 = pts_ref[0]
    lane = jax.lax.broadcasted_iota(jnp.int32, (1, pts3.shape[1]), 1)
    un = pp_ref[0, :, 0:3]
    dist = pp_ref[0, :, 3:4]
    valid = pp_ref[0, :, 4:5]
    d = jax.lax.dot_general(un, pts3, (((1,), (0,)), ((), ())),
                            preferred_element_type=jnp.float32)
    pd = jnp.abs(d + dist)
    maskf = ((pd < _THR) & (valid > 0.5) & (lane < n_valid)).astype(jnp.float32)

    x = pts3[0:1, :]
    y = pts3[1:2, :]
    z = pts3[2:3, :]
    cnt = jnp.sum(maskf, axis=1, keepdims=True)
    sx = jnp.sum(maskf * x, axis=1, keepdims=True)
    sy = jnp.sum(maskf * y, axis=1, keepdims=True)
    sz = jnp.sum(maskf * z, axis=1, keepdims=True)
    cd = jnp.maximum(cnt, 1.0)
    cx = sx / cd
    cy = sy / cd
    cz = sz / cd
    bx = ((x - cx) * maskf).astype(jnp.bfloat16).astype(jnp.float32)
    by = ((y - cy) * maskf).astype(jnp.bfloat16).astype(jnp.float32)
    bz = ((z - cz) * maskf).astype(jnp.bfloat16).astype(jnp.float32)
    cxx = jnp.sum(bx * bx, axis=1, keepdims=True)
    cyy = jnp.sum(by * by, axis=1, keepdims=True)
    czz = jnp.sum(bz * bz, axis=1, keepdims=True)
    cxy = jnp.sum(bx * by, axis=1, keepdims=True)
    cxz = jnp.sum(bx * bz, axis=1, keepdims=True)
    cyz = jnp.sum(by * bz, axis=1, keepdims=True)
    zpad = jnp.zeros((32, 118), jnp.float32)
    mom_ref[0] = jnp.concatenate(
        [cnt, sx, sy, sz, cxx, cyy, czz, cxy, cxz, cyz, zpad], axis=1)

    i = jax.lax.broadcasted_iota(jnp.int32, (1, 32), 1)
    pw_lo = jnp.where(i < 16, jnp.left_shift(1, jnp.minimum(i, 15)),
                      0).astype(jnp.float32)
    pw_hi = jnp.where(i >= 16, jnp.left_shift(1, jnp.maximum(i - 16, 0)),
                      0).astype(jnp.float32)
    lo = jax.lax.dot_general(pw_lo, maskf, (((1,), (0,)), ((), ())),
                             preferred_element_type=jnp.float32)
    hi = jax.lax.dot_general(pw_hi, maskf, (((1,), (0,)), ((), ())),
                             preferred_element_type=jnp.float32)
    zrows = jnp.zeros((6, lo.shape[1]), jnp.float32)
    bits_ref[0] = jnp.concatenate([lo, hi, zrows], axis=0)


def _sweep_kernel(pts_ref, bits_ref, pp_ref, proj_ref, disp_ref):
    X = pts_ref[0, 0:8, :]
    Y = pts_ref[0, 8:16, :]
    Z = pts_ref[0, 16:24, :]
    lo = bits_ref[0, 0:8, :].astype(jnp.uint32)
    hi = bits_ref[0, 8:16, :].astype(jnp.uint32)
    px, py, pz = X, Y, Z
    for m in range(32):
        word = lo if m < 16 else hi
        am = (jax.lax.shift_right_logical(word, jnp.uint32(m % 16))
              & jnp.uint32(1)).astype(jnp.float32)
        rnx = pp_ref[0, m:m + 1, 0:1]
        rny = pp_ref[0, m:m + 1, 1:2]
        rnz = pp_ref[0, m:m + 1, 2:3]
        rd = pp_ref[0, m:m + 1, 3:4]
        dots = rnx * px + rny * py + rnz * pz + rd
        t = am * dots
        px = px - rnx * t
        py = py - rny * t
        pz = pz - rnz * t
    proj_ref[0, 0:8, :] = px
    proj_ref[0, 8:16, :] = py
    proj_ref[0, 16:24, :] = pz
    disp_ref[0, 0:8, :] = px - X
    disp_ref[0, 8:16, :] = py - Y
    disp_ref[0, 16:24, :] = pz - Z


def kernel(points, planes):
    B, N, _ = points.shape
    M = planes.shape[1]
    pad = _NPAD - N

    pts_t = jnp.transpose(points, (0, 2, 1))
    pts_tp = jnp.pad(pts_t, ((0, 0), (0, 0), (0, pad)))

    normals = planes[:, :, :3]
    dists = planes[:, :, 3]
    norm_mag = jnp.linalg.norm(normals, axis=2)
    valid = norm_mag > 1e-6
    un = normals / jnp.maximum(norm_mag, 1e-12)[..., None]
    pp_a = jnp.concatenate(
        [un, dists[..., None], valid.astype(jnp.float32)[..., None],
         jnp.zeros((B, M, 123), jnp.float32)], axis=2)

    mom, bits_f = pl.pallas_call(
        functools.partial(_fit_kernel, n_valid=N),
        grid=(B,),
        in_specs=[
            pl.BlockSpec((1, 3, _NPAD), lambda b: (b, 0, 0)),
            pl.BlockSpec((1, M, 128), lambda b: (b, 0, 0)),
        ],
        out_specs=[
            pl.BlockSpec((1, M, 128), lambda b: (b, 0, 0)),
            pl.BlockSpec((1, 8, _NPAD), lambda b: (b, 0, 0)),
        ],
        out_shape=[
            jax.ShapeDtypeStruct((B, M, 128), jnp.float32),
            jax.ShapeDtypeStruct((B, 8, _NPAD), jnp.float32),
        ],
    )(pts_tp, pp_a)

    cnt = mom[:, :, 0]
    s = mom[:, :, 1:4]
    c = s / jnp.maximum(cnt, 1.0)[..., None]
    denom = jnp.maximum(1.0, cnt - 1.0)
    xx, yy, zz = mom[:, :, 4], mom[:, :, 5], mom[:, :, 6]
    xy, xz, yz = mom[:, :, 7], mom[:, :, 8], mom[:, :, 9]
    s2 = jnp.stack([
        jnp.stack([xx, xy, xz], -1),
        jnp.stack([xy, yy, yz], -1),
        jnp.stack([xz, yz, zz], -1)], -2)
    cov = s2 / denom[..., None, None] + 1e-6 * jnp.eye(3, dtype=jnp.float32)
    do_fit = cnt >= 3.0
    dummy = jnp.diag(jnp.array([3.0, 2.0, 1.0], dtype=jnp.float32))
    cov_safe = jnp.where(do_fit[..., None, None], cov, dummy)
    _, vee = jnp.linalg.eigh(cov_safe.reshape(B * M, 3, 3))
    rn = vee[:, 2, ::-1].reshape(B, M, 3)
    dotn = jnp.sum(rn * un, -1)
    rn = jnp.where((dotn < 0.0)[..., None], -rn, rn)
    rd = -jnp.sum(c * rn, -1)
    fitf = do_fit.astype(jnp.float32)
    rn_eff = rn * fitf[..., None]
    rd_eff = rd * fitf
    pp_b = jnp.concatenate(
        [rn_eff, rd_eff[..., None], jnp.zeros((B, M, 124), jnp.float32)],
        axis=2)

    bits_b = bits_f.reshape(B, 64, _NL)
    pts_b = pts_tp.reshape(B, 24, _NL)

    proj_r, disp_r = pl.pallas_call(
        _sweep_kernel,
        grid=(B,),
        in_specs=[
            pl.BlockSpec((1, 24, _NL), lambda b: (b, 0, 0)),
            pl.BlockSpec((1, 16, _NL), lambda b: (b, 0, 0)),
            pl.BlockSpec((1, M, 128), lambda b: (b, 0, 0)),
        ],
        out_specs=[
            pl.BlockSpec((1, 24, _NL), lambda b: (b, 0, 0)),
            pl.BlockSpec((1, 24, _NL), lambda b: (b, 0, 0)),
        ],
        out_shape=[
            jax.ShapeDtypeStruct((B, 24, _NL), jnp.float32),
            jax.ShapeDtypeStruct((B, 24, _NL), jnp.float32),
        ],
    )(pts_b, bits_b, pp_b)

    proj = jnp.transpose(proj_r.reshape(B, 3, _NPAD), (0, 2, 1))[:, :N, :]
    disp = jnp.transpose(disp_r.reshape(B, 3, _NPAD), (0, 2, 1))[:, :N, :]
    return proj, disp


# speedup vs baseline: 53.0925x; 3.6437x over previous
import functools

import jax
import jax.numpy as jnp
from jax.experimental import pallas as pl

_THR = 0.01
_NPAD = 50048
_NL = _NPAD // 8

_SWEEPS = 6
_ORDER = [(0, 2), (1, 2), (0, 1)]
_TH_CONV = 0
_VSIGN = 0


def _fit_kernel(pts_ref, pp_ref, mom_ref, bits_ref, *, n_valid):
    pts3 = pts_ref[0]
    lane = jax.lax.broadcasted_iota(jnp.int32, (1, pts3.shape[1]), 1)
    un = pp_ref[0, :, 0:3]
    dist = pp_ref[0, :, 3:4]
    valid = pp_ref[0, :, 4:5]
    d = jax.lax.dot_general(un, pts3, (((1,), (0,)), ((), ())),
                            preferred_element_type=jnp.float32)
    pd = jnp.abs(d + dist)
    maskf = ((pd < _THR) & (valid > 0.5) & (lane < n_valid)).astype(jnp.float32)

    x = pts3[0:1, :]
    y = pts3[1:2, :]
    z = pts3[2:3, :]
    cnt = jnp.sum(maskf, axis=1, keepdims=True)
    sx = jnp.sum(maskf * x, axis=1, keepdims=True)
    sy = jnp.sum(maskf * y, axis=1, keepdims=True)
    sz = jnp.sum(maskf * z, axis=1, keepdims=True)
    cd = jnp.maximum(cnt, 1.0)
    cx = sx / cd
    cy = sy / cd
    cz = sz / cd
    bx = ((x - cx) * maskf).astype(jnp.bfloat16).astype(jnp.float32)
    by = ((y - cy) * maskf).astype(jnp.bfloat16).astype(jnp.float32)
    bz = ((z - cz) * maskf).astype(jnp.bfloat16).astype(jnp.float32)
    cxx = jnp.sum(bx * bx, axis=1, keepdims=True)
    cyy = jnp.sum(by * by, axis=1, keepdims=True)
    czz = jnp.sum(bz * bz, axis=1, keepdims=True)
    cxy = jnp.sum(bx * by, axis=1, keepdims=True)
    cxz = jnp.sum(bx * bz, axis=1, keepdims=True)
    cyz = jnp.sum(by * bz, axis=1, keepdims=True)
    zpad = jnp.zeros((32, 118), jnp.float32)
    mom_ref[0] = jnp.concatenate(
        [cnt, sx, sy, sz, cxx, cyy, czz, cxy, cxz, cyz, zpad], axis=1)

    i = jax.lax.broadcasted_iota(jnp.int32, (1, 32), 1)
    pw_lo = jnp.where(i < 16, jnp.left_shift(1, jnp.minimum(i, 15)),
                      0).astype(jnp.float32)
    pw_hi = jnp.where(i >= 16, jnp.left_shift(1, jnp.maximum(i - 16, 0)),
                      0).astype(jnp.float32)
    lo = jax.lax.dot_general(pw_lo, maskf, (((1,), (0,)), ((), ())),
                             preferred_element_type=jnp.float32)
    hi = jax.lax.dot_general(pw_hi, maskf, (((1,), (0,)), ((), ())),
                             preferred_element_type=jnp.float32)
    zrows = jnp.zeros((6, lo.shape[1]), jnp.float32)
    bits_ref[0] = jnp.concatenate([lo, hi, zrows], axis=0)


def _jac_rot(a, v, p, q):
    key = (p, q) if p < q else (q, p)
    apq = a[key]
    app = a[(p, p)]
    aqq = a[(q, q)]
    num = (aqq - app) if _TH_CONV == 0 else (app - aqq)
    safe_apq = jnp.where(apq == 0.0, 1.0, apq)
    theta = num / (2.0 * safe_apq)
    sgn = jnp.where(theta < 0.0, -1.0, 1.0)
    t = sgn / (jnp.abs(theta) + jnp.sqrt(theta * theta + 1.0))
    c = 1.0 / jnp.sqrt(t * t + 1.0)
    s = t * c
    c = jnp.where(apq == 0.0, 1.0, c)
    s = jnp.where(apq == 0.0, 0.0, s)
    if _VSIGN == 1:
        s = -s
    r = ({0, 1, 2} - {p, q}).pop()
    rp = (min(r, p), max(r, p))
    rq = (min(r, q), max(r, q))
    arp = a[rp]
    arq = a[rq]
    a = dict(a)
    a[(p, p)] = c * c * app - 2.0 * s * c * apq + s * s * aqq
    a[(q, q)] = s * s * app + 2.0 * s * c * apq + c * c * aqq
    a[key] = (c * c - s * s) * apq + s * c * (app - aqq)
    a[rp] = c * arp - s * arq
    a[rq] = s * arp + c * arq
    v = dict(v)
    for i in range(3):
        vip = v[(i, p)]
        viq = v[(i, q)]
        v[(i, p)] = c * vip - s * viq
        v[(i, q)] = s * vip + c * viq
    return a, v


def _plane_kernel(min_ref, pp_ref):
    cnt = min_ref[0:1, :]
    sx = min_ref[1:2, :]
    sy = min_ref[2:3, :]
    sz = min_ref[3:4, :]
    cd = jnp.maximum(cnt, 1.0)
    cx = sx / cd
    cy = sy / cd
    cz = sz / cd
    denom = jnp.maximum(1.0, cnt - 1.0)
    fit = cnt >= 3.0
    eps = jnp.float32(1e-6)

    def cov(row, diag, dummy_val):
        val = min_ref[row:row + 1, :] / denom
        if diag:
            val = val + eps
        return jnp.where(fit, val, dummy_val)

    a = {(0, 0): cov(4, True, 3.0), (1, 1): cov(5, True, 2.0),
         (2, 2): cov(6, True, 1.0), (0, 1): cov(7, False, 0.0),
         (0, 2): cov(8, False, 0.0), (1, 2): cov(9, False, 0.0)}
    one = jnp.ones_like(cnt)
    zero = jnp.zeros_like(cnt)
    v = {(i, j): (one if i == j else zero) for i in range(3) for j in range(3)}
    for _ in range(_SWEEPS):
        for (p, q) in _ORDER:
            a, v = _jac_rot(a, v, p, q)
    w = [a[(0, 0)], a[(1, 1)], a[(2, 2)]]
    cols = [[v[(i, j)] for i in range(3)] for j in range(3)]

    def cswap(wa, ca, wb, cb):
        cond = wb < wa
        nwa = jnp.where(cond, wb, wa)
        nwb = jnp.where(cond, wa, wb)
        nca = [jnp.where(cond, cb[i], ca[i]) for i in range(3)]
        ncb = [jnp.where(cond, ca[i], cb[i]) for i in range(3)]
        return nwa, nca, nwb, ncb

    w[0], cols[0], w[1], cols[1] = cswap(w[0], cols[0], w[1], cols[1])
    w[1], cols[1], w[2], cols[2] = cswap(w[1], cols[1], w[2], cols[2])
    w[0], cols[0], w[1], cols[1] = cswap(w[0], cols[0], w[1], cols[1])

    fitf = fit.astype(jnp.float32)
    rn0 = cols[2][2] * fitf
    rn1 = cols[1][2] * fitf
    rn2 = cols[0][2] * fitf
    rd = -(cx * rn0 + cy * rn1 + cz * rn2)
    zr = jnp.zeros_like(cnt)
    pp_ref[...] = jnp.concatenate([rn0, rn1, rn2, rd, zr, zr, zr, zr], axis=0)


def _sweep_kernel(pts_ref, bits_ref, pp_ref, proj_ref, disp_ref):
    X = pts_ref[0, 0:8, :]
    Y = pts_ref[0, 8:16, :]
    Z = pts_ref[0, 16:24, :]
    lo = bits_ref[0, 0:8, :].astype(jnp.uint32)
    hi = bits_ref[0, 8:16, :].astype(jnp.uint32)
    px, py, pz = X, Y, Z
    for m in range(32):
        word = lo if m < 16 else hi
        am = (jax.lax.shift_right_logical(word, jnp.uint32(m % 16))
              & jnp.uint32(1)).astype(jnp.float32)
        rnx = pp_ref[0, 0:1, m:m + 1]
        rny = pp_ref[0, 1:2, m:m + 1]
        rnz = pp_ref[0, 2:3, m:m + 1]
        rd = pp_ref[0, 3:4, m:m + 1]
        dots = rnx * px + rny * py + rnz * pz + rd
        t = am * dots
        px = px - rnx * t
        py = py - rny * t
        pz = pz - rnz * t
    proj_ref[0, 0:8, :] = px
    proj_ref[0, 8:16, :] = py
    proj_ref[0, 16:24, :] = pz
    disp_ref[0, 0:8, :] = px - X
    disp_ref[0, 8:16, :] = py - Y
    disp_ref[0, 16:24, :] = pz - Z


def kernel(points, planes):
    B, N, _ = points.shape
    M = planes.shape[1]
    pad = _NPAD - N

    pts_t = jnp.transpose(points, (0, 2, 1))
    pts_tp = jnp.pad(pts_t, ((0, 0), (0, 0), (0, pad)))

    normals = planes[:, :, :3]
    dists = planes[:, :, 3]
    norm_mag = jnp.linalg.norm(normals, axis=2)
    valid = norm_mag > 1e-6
    un = normals / jnp.maximum(norm_mag, 1e-12)[..., None]
    pp_a = jnp.concatenate(
        [un, dists[..., None], valid.astype(jnp.float32)[..., None],
         jnp.zeros((B, M, 123), jnp.float32)], axis=2)

    mom, bits_f = pl.pallas_call(
        functools.partial(_fit_kernel, n_valid=N),
        grid=(B,),
        in_specs=[
            pl.BlockSpec((1, 3, _NPAD), lambda b: (b, 0, 0)),
            pl.BlockSpec((1, M, 128), lambda b: (b, 0, 0)),
        ],
        out_specs=[
            pl.BlockSpec((1, M, 128), lambda b: (b, 0, 0)),
            pl.BlockSpec((1, 8, _NPAD), lambda b: (b, 0, 0)),
        ],
        out_shape=[
            jax.ShapeDtypeStruct((B, M, 128), jnp.float32),
            jax.ShapeDtypeStruct((B, 8, _NPAD), jnp.float32),
        ],
    )(pts_tp, pp_a)

    min_rows = jnp.transpose(mom[:, :, :16].reshape(B * M, 16))
    pp_b = pl.pallas_call(
        _plane_kernel,
        in_specs=[pl.BlockSpec((16, B * M), lambda: (0, 0))],
        out_specs=pl.BlockSpec((8, B * M), lambda: (0, 0)),
        out_shape=jax.ShapeDtypeStruct((8, B * M), jnp.float32),
    )(min_rows)

    bits_b = bits_f.reshape(B, 64, _NL)
    pts_b = pts_tp.reshape(B, 24, _NL)
    pp_r = jnp.transpose(pp_b.reshape(8, B, M), (1, 0, 2))

    proj_r, disp_r = pl.pallas_call(
        _sweep_kernel,
        grid=(B,),
        in_specs=[
            pl.BlockSpec((1, 24, _NL), lambda b: (b, 0, 0)),
            pl.BlockSpec((1, 16, _NL), lambda b: (b, 0, 0)),
            pl.BlockSpec((1, 8, M), lambda b: (b, 0, 0)),
        ],
        out_specs=[
            pl.BlockSpec((1, 24, _NL), lambda b: (b, 0, 0)),
            pl.BlockSpec((1, 24, _NL), lambda b: (b, 0, 0)),
        ],
        out_shape=[
            jax.ShapeDtypeStruct((B, 24, _NL), jnp.float32),
            jax.ShapeDtypeStruct((B, 24, _NL), jnp.float32),
        ],
    )(pts_b, bits_b, pp_r)

    proj = jnp.transpose(proj_r.reshape(B, 3, _NPAD), (0, 2, 1))[:, :N, :]
    disp = jnp.transpose(disp_r.reshape(B, 3, _NPAD), (0, 2, 1))[:, :N, :]
    return proj, disp
